# Initial kernel scaffold; baseline (speedup 1.0000x reference)
import jax
import jax.numpy as jnp
from jax.experimental import pallas as pl


def kernel(item_indices, item_embeddings):
    raise NotImplementedError("write your pallas kernel here")


# speedup vs baseline: 1.8663x; 1.8663x over previous
import functools

import jax
import jax.numpy as jnp
from jax import lax
from jax.experimental import pallas as pl
from jax.experimental.pallas import tpu as pltpu
from jax.experimental.pallas import tpu_sc as plsc

NUM_ITEMS = 1000000
CF_DIM = 64

NC = 2
NS = 16
NW = NC * NS

B = 16384 * 50
B_PER_W = B // NW
CHUNK = 512
N_CHUNKS = B_PER_W // CHUNK
NBUF = 2
N_GROUPS = N_CHUNKS // NBUF


def _sc_gather(idx_flat, table):
  mesh = plsc.VectorSubcoreMesh(core_axis_name="c", subcore_axis_name="s")

  @functools.partial(
      pl.kernel,
      out_type=jax.ShapeDtypeStruct((B, CF_DIM), jnp.float32),
      mesh=mesh,
      compiler_params=pltpu.CompilerParams(use_tc_tiling_on_sc=False),
      scratch_types=[
          pltpu.VMEM((N_CHUNKS, CHUNK), jnp.int32),
          *[pltpu.VMEM((CHUNK, CF_DIM), jnp.float32) for _ in range(NBUF)],
          *[pltpu.SemaphoreType.DMA for _ in range(2 * NBUF)],
      ],
  )
  def k(idx_hbm, table_hbm, out_hbm, idx_v, *bufs_and_sems):
    rows = bufs_and_sems[:NBUF]
    gsem = bufs_and_sems[NBUF:2 * NBUF]
    osem = bufs_and_sems[2 * NBUF:]
    wid = lax.axis_index("s") * NC + lax.axis_index("c")
    base = wid * B_PER_W

    pltpu.sync_copy(idx_hbm.at[wid], idx_v)

    @pl.loop(0, N_GROUPS)
    def _(g):
      for b in range(NBUF):
        c = g * NBUF + b

        @pl.when(g > 0)
        def _():
          pltpu.make_async_copy(
              rows[b], out_hbm.at[pl.ds(base, CHUNK)], osem[b]).wait()

        pltpu.async_copy(table_hbm.at[idx_v.at[c]], rows[b], gsem[b])

      for b in range(NBUF):
        c = g * NBUF + b
        pltpu.make_async_copy(
            table_hbm.at[pl.ds(0, CHUNK)], rows[b], gsem[b]).wait()
        pltpu.async_copy(
            rows[b], out_hbm.at[pl.ds(base + c * CHUNK, CHUNK)], osem[b])

    for b in range(NBUF):
      pltpu.make_async_copy(
          rows[b], out_hbm.at[pl.ds(base, CHUNK)], osem[b]).wait()

  return k(idx_flat, table)


@jax.jit
def kernel(item_indices, item_embeddings):
  idx = item_indices.astype(jnp.int32).reshape(NW, N_CHUNKS, CHUNK)
  out = _sc_gather(idx, item_embeddings)
  return out.reshape(item_indices.shape + (CF_DIM,))


# speedup vs baseline: 1.8699x; 1.0020x over previous
import functools

import jax
import jax.numpy as jnp
from jax import lax
from jax.experimental import pallas as pl
from jax.experimental.pallas import tpu as pltpu
from jax.experimental.pallas import tpu_sc as plsc

NUM_ITEMS = 1000000
CF_DIM = 64

NC = 2
NS = 16
NW = NC * NS

B = 16384 * 50
B_PER_W = B // NW
CHUNK = 256
N_CHUNKS = B_PER_W // CHUNK
NBUF = 4
N_GROUPS = N_CHUNKS // NBUF


def _sc_gather(idx_flat, table):
  mesh = plsc.VectorSubcoreMesh(core_axis_name="c", subcore_axis_name="s")

  @functools.partial(
      pl.kernel,
      out_type=jax.ShapeDtypeStruct((B, CF_DIM), jnp.float32),
      mesh=mesh,
      compiler_params=pltpu.CompilerParams(use_tc_tiling_on_sc=False),
      scratch_types=[
          pltpu.VMEM((N_CHUNKS, CHUNK), jnp.int32),
          *[pltpu.VMEM((CHUNK, CF_DIM), jnp.float32) for _ in range(NBUF)],
          *[pltpu.SemaphoreType.DMA for _ in range(2 * NBUF)],
      ],
  )
  def k(idx_hbm, table_hbm, out_hbm, idx_v, *bufs_and_sems):
    rows = bufs_and_sems[:NBUF]
    gsem = bufs_and_sems[NBUF:2 * NBUF]
    osem = bufs_and_sems[2 * NBUF:]
    wid = lax.axis_index("s") * NC + lax.axis_index("c")
    base = wid * B_PER_W

    pltpu.sync_copy(idx_hbm.at[wid], idx_v)

    @pl.loop(0, N_GROUPS)
    def _(g):
      for b in range(NBUF):
        c = g * NBUF + b

        @pl.when(g > 0)
        def _():
          pltpu.make_async_copy(
              rows[b], out_hbm.at[pl.ds(base, CHUNK)], osem[b]).wait()

        pltpu.async_copy(table_hbm.at[idx_v.at[c]], rows[b], gsem[b])

      for b in range(NBUF):
        c = g * NBUF + b
        pltpu.make_async_copy(
            table_hbm.at[pl.ds(0, CHUNK)], rows[b], gsem[b]).wait()
        pltpu.async_copy(
            rows[b], out_hbm.at[pl.ds(base + c * CHUNK, CHUNK)], osem[b])

    for b in range(NBUF):
      pltpu.make_async_copy(
          rows[b], out_hbm.at[pl.ds(base, CHUNK)], osem[b]).wait()

  return k(idx_flat, table)


@jax.jit
def kernel(item_indices, item_embeddings):
  idx = item_indices.astype(jnp.int32).reshape(NW, N_CHUNKS, CHUNK)
  out = _sc_gather(idx, item_embeddings)
  return out.reshape(item_indices.shape + (CF_DIM,))


# speedup vs baseline: 2.3214x; 1.2414x over previous
import functools

import jax
import jax.numpy as jnp
from jax import lax
from jax.experimental import pallas as pl
from jax.experimental.pallas import tpu as pltpu
from jax.experimental.pallas import tpu_sc as plsc
from jax.experimental import layout as jax_layout

NUM_ITEMS = 1000000
CF_DIM = 64

NC = 2
NS = 16
NW = NC * NS

B = 16384 * 50
B_PER_W = B // NW
CHUNK = 256
N_CHUNKS = B_PER_W // CHUNK
NBUF = 4
N_GROUPS = N_CHUNKS // NBUF


def _sc_gather(idx_flat, table):
  mesh = plsc.VectorSubcoreMesh(core_axis_name="c", subcore_axis_name="s")

  @functools.partial(
      pl.kernel,
      out_type=jax.ShapeDtypeStruct((B, CF_DIM), jnp.float32),
      mesh=mesh,
      compiler_params=pltpu.CompilerParams(use_tc_tiling_on_sc=False),
      scratch_types=[
          pltpu.VMEM((N_CHUNKS, CHUNK), jnp.int32),
          *[pltpu.VMEM((CHUNK, CF_DIM), jnp.float32) for _ in range(NBUF)],
          *[pltpu.SemaphoreType.DMA for _ in range(2 * NBUF)],
      ],
  )
  def k(idx_hbm, table_hbm, out_hbm, idx_v, *bufs_and_sems):
    rows = bufs_and_sems[:NBUF]
    gsem = bufs_and_sems[NBUF:2 * NBUF]
    osem = bufs_and_sems[2 * NBUF:]
    wid = lax.axis_index("s") * NC + lax.axis_index("c")
    base = wid * B_PER_W

    pltpu.sync_copy(idx_hbm.at[wid], idx_v)

    @pl.loop(0, N_GROUPS)
    def _(g):
      for b in range(NBUF):
        c = g * NBUF + b

        @pl.when(g > 0)
        def _():
          pltpu.make_async_copy(
              rows[b], out_hbm.at[pl.ds(base, CHUNK)], osem[b]).wait()

        pltpu.async_copy(table_hbm.at[idx_v.at[c]], rows[b], gsem[b])

      for b in range(NBUF):
        c = g * NBUF + b
        pltpu.make_async_copy(
            table_hbm.at[pl.ds(0, CHUNK)], rows[b], gsem[b]).wait()
        pltpu.async_copy(
            rows[b], out_hbm.at[pl.ds(base + c * CHUNK, CHUNK)], osem[b])

    for b in range(NBUF):
      pltpu.make_async_copy(
          rows[b], out_hbm.at[pl.ds(base, CHUNK)], osem[b]).wait()

  return k(idx_flat, table)


@jax.jit
def kernel(item_indices, item_embeddings):
  idx = item_indices.astype(jnp.int32).reshape(NW, N_CHUNKS, CHUNK)
  table = jax_layout.with_layout_constraint(
      item_embeddings,
      jax_layout.Layout((0, 1), tiling=((8,), (1024,))),
  )
  out = _sc_gather(idx, table)
  return out.reshape(item_indices.shape + (CF_DIM,))
